# baseline (speedup 1.0000x reference)
import functools

import jax
import jax.numpy as jnp
from jax import lax
from jax.experimental import pallas as pl
from jax.experimental.pallas import tpu as pltpu
from jax.experimental.pallas import tpu_sc as plsc

N = 10000
E = 320000
DIN = 128
DH = 32
STATE_DIM = 64
LSTM_H = 256
ACT_DIM = 64

NC = 2
NS = 16
NW = NC * NS
CHUNK = 128
NCH = 80
E_TILE = NCH * CHUNK
E_PAD = NW * E_TILE
N_PAD = 10240
STRIPE = N_PAD // NS

_MESH = plsc.VectorSubcoreMesh(core_axis_name="c", subcore_axis_name="s")
_SC_PARAMS = pltpu.CompilerParams(use_tc_tiling_on_sc=False,
                                  needs_layout_passes=False)


def _rsqrt16(x):
    i = plsc.bitcast(x, jnp.int32)
    i = 0x5F3759DF - (i >> 1)
    y = plsc.bitcast(i, jnp.float32)
    for _ in range(3):
        y = y * (1.5 - 0.5 * x * y * y)
    return y


@functools.partial(
    pl.kernel,
    out_type=jax.ShapeDtypeStruct((NC, N_PAD), jnp.float32),
    mesh=_MESH,
    scratch_types=[
        pltpu.VMEM((NCH, CHUNK), jnp.int32),
        pltpu.VMEM((NCH, CHUNK), jnp.float32),
        pltpu.VMEM_SHARED((N_PAD,), jnp.float32),
        pltpu.SemaphoreType.DMA,
    ],
    compiler_params=_SC_PARAMS,
)
def _deg_call(dsts_hbm, ews_hbm, zeros_hbm, out_hbm, dst_v, ew_v, deg_sh,
              dsem):
    cid = lax.axis_index("c")
    sid = lax.axis_index("s")
    wid = cid * NS + sid
    pltpu.sync_copy(zeros_hbm.at[pl.ds(sid * STRIPE, STRIPE)],
                    deg_sh.at[pl.ds(sid * STRIPE, STRIPE)])
    pltpu.sync_copy(dsts_hbm.at[wid], dst_v)
    pltpu.sync_copy(ews_hbm.at[wid], ew_v)
    plsc.subcore_barrier()

    def body(j, _):
        pltpu.async_copy(ew_v.at[j], deg_sh.at[dst_v.at[j]], dsem, add=True)
        return 0

    lax.fori_loop(0, NCH, body, 0)

    def drain(j, _):
        pltpu.make_async_copy(ew_v.at[j], deg_sh.at[dst_v.at[j]], dsem).wait()
        return 0

    lax.fori_loop(0, NCH, drain, 0)
    plsc.subcore_barrier()
    pltpu.sync_copy(deg_sh.at[pl.ds(sid * STRIPE, STRIPE)],
                    out_hbm.at[cid, pl.ds(sid * STRIPE, STRIPE)])


def _msg_phases(g_hbm, srcs_hbm, dsts_hbm, zeros_hbm, acc_hbm,
                src_v, dst_v, ew_v, rows, acc_sh, gsem, ssem, cid, sid, wid):
    def start_gather(j, b):
        pltpu.make_async_copy(g_hbm.at[src_v.at[j]], rows[b], gsem[b]).start()

    def wait_gather(j, b):
        pltpu.make_async_copy(g_hbm.at[src_v.at[j]], rows[b], gsem[b]).wait()

    def start_scatter(j, b):
        pltpu.async_copy(rows[b], acc_sh.at[dst_v.at[j]], ssem[b], add=True)

    def wait_scatter(j, b):
        pltpu.make_async_copy(rows[b], acc_sh.at[dst_v.at[j]], ssem[b]).wait()

    def scale(b, j):
        buf = rows[b]
        for k in range(CHUNK // 16):
            wv16 = ew_v[j, pl.ds(16 * k, 16)]
            for l in range(16):
                i = 16 * k + l
                wv = jnp.full((16,), wv16[l], dtype=jnp.float32)
                buf[i, 0:16] = buf[i, 0:16] * wv
                buf[i, 16:32] = buf[i, 16:32] * wv

    start_gather(0, 0)
    start_gather(1, 1)

    def body(t, _):
        for b in range(5):
            j = 5 * t + b
            wait_gather(j, b)
            scale(b, j)
            start_scatter(j, b)

            @pl.when(j >= 3)
            def _():
                wait_scatter(j - 3, (b - 3) % 5)

            @pl.when(j + 2 < NCH)
            def _():
                start_gather(j + 2, (b + 2) % 5)

        return 0

    lax.fori_loop(0, NCH // 5, body, 0)
    wait_scatter(NCH - 3, (NCH - 3) % 5)
    wait_scatter(NCH - 2, (NCH - 2) % 5)
    wait_scatter(NCH - 1, (NCH - 1) % 5)
    plsc.subcore_barrier()
    pltpu.sync_copy(acc_sh.at[pl.ds(sid * STRIPE, STRIPE)],
                    acc_hbm.at[cid, pl.ds(sid * STRIPE, STRIPE)])


@functools.partial(
    pl.kernel,
    out_type=[
        jax.ShapeDtypeStruct((NC, N_PAD, DH), jnp.float32),
        jax.ShapeDtypeStruct((NW, NCH, CHUNK), jnp.float32),
    ],
    mesh=_MESH,
    scratch_types=[
        pltpu.VMEM((NCH, CHUNK), jnp.int32),
        pltpu.VMEM((NCH, CHUNK), jnp.int32),
        pltpu.VMEM((NCH, CHUNK), jnp.float32),
        [pltpu.VMEM((CHUNK, DH), jnp.float32) for _ in range(5)],
        pltpu.VMEM((N_PAD,), jnp.float32),
        pltpu.VMEM((STRIPE,), jnp.float32),
        pltpu.VMEM((STRIPE,), jnp.float32),
        pltpu.VMEM_SHARED((N_PAD, DH), jnp.float32),
        pltpu.VMEM_SHARED((N_PAD,), jnp.float32),
        [pltpu.SemaphoreType.DMA for _ in range(5)],
        [pltpu.SemaphoreType.DMA for _ in range(5)],
    ],
    compiler_params=_SC_PARAMS,
)
def _msg1_call(g_hbm, srcs_hbm, dsts_hbm, ews_hbm, degp_hbm, zeros_hbm,
               acc_hbm, norm_hbm, src_v, dst_v, ew_v, rows, dtab, d0, d1,
               acc_sh, dis_sh, gsem, ssem):
    cid = lax.axis_index("c")
    sid = lax.axis_index("s")
    wid = cid * NS + sid

    pltpu.sync_copy(zeros_hbm.at[pl.ds(sid * STRIPE, STRIPE)],
                    acc_sh.at[pl.ds(sid * STRIPE, STRIPE)])
    pltpu.sync_copy(srcs_hbm.at[wid], src_v)
    pltpu.sync_copy(dsts_hbm.at[wid], dst_v)
    pltpu.sync_copy(ews_hbm.at[wid], ew_v)
    pltpu.sync_copy(degp_hbm.at[0, pl.ds(sid * STRIPE, STRIPE)], d0)
    pltpu.sync_copy(degp_hbm.at[1, pl.ds(sid * STRIPE, STRIPE)], d1)
    for v in range(STRIPE // 16):
        s = pl.ds(16 * v, 16)
        d0[s] = _rsqrt16(d0[s] + d1[s] + 1.0)
    pltpu.sync_copy(d0, dis_sh.at[pl.ds(sid * STRIPE, STRIPE)])
    plsc.subcore_barrier()
    pltpu.sync_copy(dis_sh, dtab)
    def norm_body(j, _):
        for k in range(CHUNK // 16):
            s = pl.ds(16 * k, 16)
            a = plsc.load_gather(dtab, [src_v[j, s]])
            b = plsc.load_gather(dtab, [dst_v[j, s]])
            ew_v[j, s] = a * b * ew_v[j, s]
        return 0

    lax.fori_loop(0, NCH, norm_body, 0)
    pltpu.sync_copy(ew_v, norm_hbm.at[wid])

    _msg_phases(g_hbm, srcs_hbm, dsts_hbm, zeros_hbm, acc_hbm,
                src_v, dst_v, ew_v, rows, acc_sh, gsem, ssem, cid, sid, wid)


@functools.partial(
    pl.kernel,
    out_type=jax.ShapeDtypeStruct((NC, N_PAD, DH), jnp.float32),
    mesh=_MESH,
    scratch_types=[
        pltpu.VMEM((NCH, CHUNK), jnp.int32),
        pltpu.VMEM((NCH, CHUNK), jnp.int32),
        pltpu.VMEM((NCH, CHUNK), jnp.float32),
        [pltpu.VMEM((CHUNK, DH), jnp.float32) for _ in range(5)],
        pltpu.VMEM_SHARED((N_PAD, DH), jnp.float32),
        [pltpu.SemaphoreType.DMA for _ in range(5)],
        [pltpu.SemaphoreType.DMA for _ in range(5)],
    ],
    compiler_params=_SC_PARAMS,
)
def _msg2_call(g_hbm, srcs_hbm, dsts_hbm, norms_hbm, zeros_hbm, acc_hbm,
               src_v, dst_v, ew_v, rows, acc_sh, gsem, ssem):
    cid = lax.axis_index("c")
    sid = lax.axis_index("s")
    wid = cid * NS + sid

    pltpu.sync_copy(zeros_hbm.at[pl.ds(sid * STRIPE, STRIPE)],
                    acc_sh.at[pl.ds(sid * STRIPE, STRIPE)])
    pltpu.sync_copy(srcs_hbm.at[wid], src_v)
    pltpu.sync_copy(dsts_hbm.at[wid], dst_v)
    pltpu.sync_copy(norms_hbm.at[wid], ew_v)
    plsc.subcore_barrier()

    _msg_phases(g_hbm, srcs_hbm, dsts_hbm, zeros_hbm, acc_hbm,
                src_v, dst_v, ew_v, rows, acc_sh, gsem, ssem, cid, sid, wid)


NQ = N // 4


def _dense_a_body(x_ref, w1_ref, h1_ref):
    w1 = w1_ref[...]
    parts = [jnp.dot(x_ref[pl.ds(q * NQ, NQ), :], w1,
                     preferred_element_type=jnp.float32) for q in range(4)]
    h1_ref[...] = jnp.concatenate(parts, axis=1)


_dense_a = pl.pallas_call(
    _dense_a_body,
    out_shape=jax.ShapeDtypeStruct((NQ, 4 * DH), jnp.float32),
)


def _inv_packed(degp4_ref):
    dm = (degp4_ref[0, pl.ds(0, NQ), :] + degp4_ref[1, pl.ds(0, NQ), :]
          + 1.0)
    ci = lax.broadcasted_iota(jnp.int32, (4, 4 * DH), 1) // DH
    ri = lax.broadcasted_iota(jnp.int32, (4, 4 * DH), 0)
    exp = (ci == ri).astype(jnp.float32)
    return jnp.dot(1.0 / dm, exp, preferred_element_type=jnp.float32)


def _acc2500(accp_ref):
    return accp_ref[0, pl.ds(0, NQ), :] + accp_ref[1, pl.ds(0, NQ), :]


def _dense_b_body(accp_ref, h_ref, degp4_ref, b_ref, w2bd_ref, h2_ref):
    x1 = jnp.maximum(
        _acc2500(accp_ref) + h_ref[...] * _inv_packed(degp4_ref)
        + b_ref[...], 0.0)
    h2_ref[...] = jnp.dot(x1, w2bd_ref[...],
                          preferred_element_type=jnp.float32)


_dense_b = pl.pallas_call(
    _dense_b_body,
    out_shape=jax.ShapeDtypeStruct((NQ, 4 * DH), jnp.float32),
)


def _dense_c_body(accp_ref, h_ref, degp4_ref, b_ref, xs_ref, h0_ref,
                  c0_ref, wih_ref, whh_ref, bih_ref, bhh_ref, wfc_ref,
                  bfc_ref, xo_ref, h1o_ref, c1o_ref):
    x2 = jnp.maximum(
        _acc2500(accp_ref) + h_ref[...] * _inv_packed(degp4_ref)
        + b_ref[...], 0.0)
    s128 = jnp.sum(x2, axis=0, keepdims=True)
    if True:
        xg = (s128[:, 0:DH] + s128[:, DH:2 * DH] + s128[:, 2 * DH:3 * DH]
              + s128[:, 3 * DH:4 * DH]) * (1.0 / N)
        xc = jnp.concatenate([xg, xs_ref[...]], axis=1)
        cdims = (((1,), (1,)), ((), ()))
        gates = (lax.dot_general(xc, wih_ref[...], cdims,
                                 preferred_element_type=jnp.float32)
                 + bih_ref[...]
                 + lax.dot_general(h0_ref[...], whh_ref[...], cdims,
                                   preferred_element_type=jnp.float32)
                 + bhh_ref[...])
        H = LSTM_H
        gi = jax.nn.sigmoid(gates[:, 0:H])
        gf = jax.nn.sigmoid(gates[:, H:2 * H])
        gg = jnp.tanh(gates[:, 2 * H:3 * H])
        go = jax.nn.sigmoid(gates[:, 3 * H:4 * H])
        c1 = gf * c0_ref[...] + gi * gg
        h1 = go * jnp.tanh(c1)
        logits = lax.dot_general(h1, wfc_ref[...], cdims,
                                 preferred_element_type=jnp.float32) + bfc_ref[...]
        m = jnp.max(logits, axis=1, keepdims=True)
        lse = jnp.log(jnp.sum(jnp.exp(logits - m), axis=1, keepdims=True))
        xo_ref[...] = (logits - m - lse).reshape(1, 1, ACT_DIM)
        h1o_ref[...] = h1.reshape(1, 1, LSTM_H)
        c1o_ref[...] = c1.reshape(1, 1, LSTM_H)


_dense_c = pl.pallas_call(
    _dense_c_body,
    out_shape=[
        jax.ShapeDtypeStruct((1, 1, ACT_DIM), jnp.float32),
        jax.ShapeDtypeStruct((1, 1, LSTM_H), jnp.float32),
        jax.ShapeDtypeStruct((1, 1, LSTM_H), jnp.float32),
    ],
)


def kernel(x_graph, edge_index, edge_weight, x_state, h0, c0, W1, b1, W2, b2,
           W_ih, W_hh, b_ih, b_hh, W_fc, b_fc):
    npad = E_PAD - E
    em = 4 * edge_index - 9999 * (edge_index // NQ)
    pad_src = jnp.arange(npad, dtype=jnp.int32) % N
    pad_dst = jnp.arange(npad, dtype=jnp.int32) % (N_PAD - N) + N
    srcs = jnp.concatenate([em[0], pad_src]).reshape(NW, NCH, CHUNK)
    dsts = jnp.concatenate([em[1], pad_dst]).reshape(NW, NCH, CHUNK)
    ews = jnp.concatenate(
        [edge_weight, jnp.zeros((npad,), jnp.float32)]).reshape(NW, NCH, CHUNK)
    zeros1 = jnp.zeros((N_PAD,), jnp.float32)
    zeros2 = jnp.zeros((N_PAD, DH), jnp.float32)
    w2bd = jnp.kron(jnp.eye(4, dtype=jnp.float32), W2)

    degp = _deg_call(dsts, ews, zeros1)
    degp4 = degp.reshape(NC, N_PAD // 4, 4)
    h1 = _dense_a(x_graph, W1)
    accp1, norms = _msg1_call(h1.reshape(N, DH), srcs, dsts, ews, degp,
                              zeros2)
    h2 = _dense_b(accp1.reshape(NC, N_PAD // 4, 4 * DH), h1, degp4,
                  jnp.tile(b1.reshape(1, DH), (1, 4)), w2bd)
    accp2 = _msg2_call(h2.reshape(N, DH), srcs, dsts, norms, zeros2)
    xo, h1o, c1o = _dense_c(accp2.reshape(NC, N_PAD // 4, 4 * DH), h2, degp4,
                            jnp.tile(b2.reshape(1, DH), (1, 4)), x_state,
                            h0.reshape(1, LSTM_H), c0.reshape(1, LSTM_H),
                            W_ih, W_hh, b_ih.reshape(1, 4 * LSTM_H),
                            b_hh.reshape(1, 4 * LSTM_H), W_fc,
                            b_fc.reshape(1, ACT_DIM))
    return (xo, h1o, c1o)


# speedup vs baseline: 1.0946x; 1.0946x over previous
import functools

import jax
import jax.numpy as jnp
from jax import lax
from jax.experimental import pallas as pl
from jax.experimental.pallas import tpu as pltpu
from jax.experimental.pallas import tpu_sc as plsc

N = 10000
E = 320000
DIN = 128
DH = 32
STATE_DIM = 64
LSTM_H = 256
ACT_DIM = 64

NC = 2
NS = 16
NW = NC * NS
CHUNK = 128
NCH = 80
E_TILE = NCH * CHUNK
E_PAD = NW * E_TILE
N_PAD = 10240
STRIPE = N_PAD // NS

_MESH = plsc.VectorSubcoreMesh(core_axis_name="c", subcore_axis_name="s")
_SC_PARAMS = pltpu.CompilerParams(use_tc_tiling_on_sc=False,
                                  needs_layout_passes=False)


def _rsqrt16(x):
    i = plsc.bitcast(x, jnp.int32)
    i = 0x5F3759DF - (i >> 1)
    y = plsc.bitcast(i, jnp.float32)
    for _ in range(3):
        y = y * (1.5 - 0.5 * x * y * y)
    return y


@functools.partial(
    pl.kernel,
    out_type=jax.ShapeDtypeStruct((NC, N_PAD), jnp.float32),
    mesh=_MESH,
    scratch_types=[
        pltpu.VMEM((NCH, CHUNK), jnp.int32),
        pltpu.VMEM((NCH, CHUNK), jnp.float32),
        pltpu.VMEM_SHARED((N_PAD,), jnp.float32),
        pltpu.SemaphoreType.DMA,
    ],
    compiler_params=_SC_PARAMS,
)
def _deg_call(dsts_hbm, ews_hbm, zeros_hbm, out_hbm, dst_v, ew_v, deg_sh,
              dsem):
    cid = lax.axis_index("c")
    sid = lax.axis_index("s")
    wid = cid * NS + sid
    pltpu.sync_copy(zeros_hbm.at[pl.ds(sid * STRIPE, STRIPE)],
                    deg_sh.at[pl.ds(sid * STRIPE, STRIPE)])
    pltpu.sync_copy(dsts_hbm.at[wid], dst_v)
    pltpu.sync_copy(ews_hbm.at[wid], ew_v)
    plsc.subcore_barrier()

    def body(j, _):
        pltpu.async_copy(ew_v.at[j], deg_sh.at[dst_v.at[j]], dsem, add=True)
        return 0

    lax.fori_loop(0, NCH, body, 0)

    def drain(j, _):
        pltpu.make_async_copy(ew_v.at[j], deg_sh.at[dst_v.at[j]], dsem).wait()
        return 0

    lax.fori_loop(0, NCH, drain, 0)
    plsc.subcore_barrier()
    pltpu.sync_copy(deg_sh.at[pl.ds(sid * STRIPE, STRIPE)],
                    out_hbm.at[cid, pl.ds(sid * STRIPE, STRIPE)])


def _msg_phases(g_hbm, srcs_hbm, dsts_hbm, zeros_hbm, acc_hbm,
                src_v, dst_v, ew_v, rows, acc_sh, gsem, ssem, cid, sid, wid):
    def start_gather(j, b):
        pltpu.make_async_copy(g_hbm.at[src_v.at[j]], rows[b], gsem[b]).start()

    def wait_gather(j, b):
        pltpu.make_async_copy(g_hbm.at[src_v.at[j]], rows[b], gsem[b]).wait()

    def start_scatter(j, b):
        pltpu.async_copy(rows[b], acc_sh.at[dst_v.at[j]], ssem[b], add=True)

    def wait_scatter(j, b):
        pltpu.make_async_copy(rows[b], acc_sh.at[dst_v.at[j]], ssem[b]).wait()

    def scale(b, j):
        buf = rows[b]
        for k in range(CHUNK // 16):
            wv16 = ew_v[j, pl.ds(16 * k, 16)]
            for l in range(16):
                i = 16 * k + l
                wv = jnp.full((16,), wv16[l], dtype=jnp.float32)
                buf[i, 0:16] = buf[i, 0:16] * wv
                buf[i, 16:32] = buf[i, 16:32] * wv

    start_gather(0, 0)
    start_gather(1, 1)

    def body(t, _):
        for b in range(4):
            j = 4 * t + b
            wait_gather(j, b)
            scale(b, j)
            start_scatter(j, b)

            @pl.when(j >= 2)
            def _():
                wait_scatter(j - 2, (b - 2) % 4)

            @pl.when(j + 2 < NCH)
            def _():
                start_gather(j + 2, (b + 2) % 4)

        return 0

    lax.fori_loop(0, NCH // 4, body, 0)
    wait_scatter(NCH - 2, (NCH - 2) % 4)
    wait_scatter(NCH - 1, (NCH - 1) % 4)
    plsc.subcore_barrier()
    pltpu.sync_copy(acc_sh.at[pl.ds(sid * STRIPE, STRIPE)],
                    acc_hbm.at[cid, pl.ds(sid * STRIPE, STRIPE)])


@functools.partial(
    pl.kernel,
    out_type=[
        jax.ShapeDtypeStruct((NC, N_PAD, DH), jnp.float32),
        jax.ShapeDtypeStruct((NW, NCH, CHUNK), jnp.float32),
    ],
    mesh=_MESH,
    scratch_types=[
        pltpu.VMEM((NCH, CHUNK), jnp.int32),
        pltpu.VMEM((NCH, CHUNK), jnp.int32),
        pltpu.VMEM((NCH, CHUNK), jnp.float32),
        [pltpu.VMEM((CHUNK, DH), jnp.float32) for _ in range(4)],
        pltpu.VMEM((N_PAD,), jnp.float32),
        pltpu.VMEM((STRIPE,), jnp.float32),
        pltpu.VMEM((STRIPE,), jnp.float32),
        pltpu.VMEM_SHARED((N_PAD, DH), jnp.float32),
        pltpu.VMEM_SHARED((N_PAD,), jnp.float32),
        [pltpu.SemaphoreType.DMA for _ in range(4)],
        [pltpu.SemaphoreType.DMA for _ in range(4)],
    ],
    compiler_params=_SC_PARAMS,
)
def _msg1_call(g_hbm, srcs_hbm, dsts_hbm, ews_hbm, degp_hbm, zeros_hbm,
               acc_hbm, norm_hbm, src_v, dst_v, ew_v, rows, dtab, d0, d1,
               acc_sh, dis_sh, gsem, ssem):
    cid = lax.axis_index("c")
    sid = lax.axis_index("s")
    wid = cid * NS + sid

    pltpu.sync_copy(zeros_hbm.at[pl.ds(sid * STRIPE, STRIPE)],
                    acc_sh.at[pl.ds(sid * STRIPE, STRIPE)])
    pltpu.sync_copy(srcs_hbm.at[wid], src_v)
    pltpu.sync_copy(dsts_hbm.at[wid], dst_v)
    pltpu.sync_copy(ews_hbm.at[wid], ew_v)
    pltpu.sync_copy(degp_hbm.at[0, pl.ds(sid * STRIPE, STRIPE)], d0)
    pltpu.sync_copy(degp_hbm.at[1, pl.ds(sid * STRIPE, STRIPE)], d1)
    for v in range(STRIPE // 16):
        s = pl.ds(16 * v, 16)
        d0[s] = _rsqrt16(d0[s] + d1[s] + 1.0)
    pltpu.sync_copy(d0, dis_sh.at[pl.ds(sid * STRIPE, STRIPE)])
    plsc.subcore_barrier()
    pltpu.sync_copy(dis_sh, dtab)
    def norm_body(j, _):
        for k in range(CHUNK // 16):
            s = pl.ds(16 * k, 16)
            a = plsc.load_gather(dtab, [src_v[j, s]])
            b = plsc.load_gather(dtab, [dst_v[j, s]])
            ew_v[j, s] = a * b * ew_v[j, s]
        return 0

    lax.fori_loop(0, NCH, norm_body, 0)
    pltpu.sync_copy(ew_v, norm_hbm.at[wid])

    _msg_phases(g_hbm, srcs_hbm, dsts_hbm, zeros_hbm, acc_hbm,
                src_v, dst_v, ew_v, rows, acc_sh, gsem, ssem, cid, sid, wid)


@functools.partial(
    pl.kernel,
    out_type=jax.ShapeDtypeStruct((NC, N_PAD, DH), jnp.float32),
    mesh=_MESH,
    scratch_types=[
        pltpu.VMEM((NCH, CHUNK), jnp.int32),
        pltpu.VMEM((NCH, CHUNK), jnp.int32),
        pltpu.VMEM((NCH, CHUNK), jnp.float32),
        [pltpu.VMEM((CHUNK, DH), jnp.float32) for _ in range(4)],
        pltpu.VMEM_SHARED((N_PAD, DH), jnp.float32),
        [pltpu.SemaphoreType.DMA for _ in range(4)],
        [pltpu.SemaphoreType.DMA for _ in range(4)],
    ],
    compiler_params=_SC_PARAMS,
)
def _msg2_call(g_hbm, srcs_hbm, dsts_hbm, norms_hbm, zeros_hbm, acc_hbm,
               src_v, dst_v, ew_v, rows, acc_sh, gsem, ssem):
    cid = lax.axis_index("c")
    sid = lax.axis_index("s")
    wid = cid * NS + sid

    pltpu.sync_copy(zeros_hbm.at[pl.ds(sid * STRIPE, STRIPE)],
                    acc_sh.at[pl.ds(sid * STRIPE, STRIPE)])
    pltpu.sync_copy(srcs_hbm.at[wid], src_v)
    pltpu.sync_copy(dsts_hbm.at[wid], dst_v)
    pltpu.sync_copy(norms_hbm.at[wid], ew_v)
    plsc.subcore_barrier()

    _msg_phases(g_hbm, srcs_hbm, dsts_hbm, zeros_hbm, acc_hbm,
                src_v, dst_v, ew_v, rows, acc_sh, gsem, ssem, cid, sid, wid)


NQ = N // 4


def _dense_a_body(x_ref, w1_ref, h1_ref):
    w1 = w1_ref[...]
    parts = [jnp.dot(x_ref[pl.ds(q * NQ, NQ), :], w1,
                     preferred_element_type=jnp.float32) for q in range(4)]
    h1_ref[...] = jnp.concatenate(parts, axis=1)


_dense_a = pl.pallas_call(
    _dense_a_body,
    out_shape=jax.ShapeDtypeStruct((NQ, 4 * DH), jnp.float32),
)


def _inv_packed(degp4_ref):
    dm = (degp4_ref[0, pl.ds(0, NQ), :] + degp4_ref[1, pl.ds(0, NQ), :]
          + 1.0)
    ci = lax.broadcasted_iota(jnp.int32, (4, 4 * DH), 1) // DH
    ri = lax.broadcasted_iota(jnp.int32, (4, 4 * DH), 0)
    exp = (ci == ri).astype(jnp.float32)
    return jnp.dot(1.0 / dm, exp, preferred_element_type=jnp.float32)


def _acc2500(accp_ref):
    return accp_ref[0, pl.ds(0, NQ), :] + accp_ref[1, pl.ds(0, NQ), :]


def _dense_b_body(accp_ref, h_ref, degp4_ref, b_ref, w2bd_ref, h2_ref):
    x1 = jnp.maximum(
        _acc2500(accp_ref) + h_ref[...] * _inv_packed(degp4_ref)
        + b_ref[...], 0.0)
    h2_ref[...] = jnp.dot(x1, w2bd_ref[...],
                          preferred_element_type=jnp.float32)


_dense_b = pl.pallas_call(
    _dense_b_body,
    out_shape=jax.ShapeDtypeStruct((NQ, 4 * DH), jnp.float32),
)


def _dense_c_body(accp_ref, h_ref, degp4_ref, b_ref, xs_ref, h0_ref,
                  c0_ref, wih_ref, whh_ref, bih_ref, bhh_ref, wfc_ref,
                  bfc_ref, xo_ref, h1o_ref, c1o_ref):
    x2 = jnp.maximum(
        _acc2500(accp_ref) + h_ref[...] * _inv_packed(degp4_ref)
        + b_ref[...], 0.0)
    s128 = jnp.sum(x2, axis=0, keepdims=True)
    if True:
        xg = (s128[:, 0:DH] + s128[:, DH:2 * DH] + s128[:, 2 * DH:3 * DH]
              + s128[:, 3 * DH:4 * DH]) * (1.0 / N)
        xc = jnp.concatenate([xg, xs_ref[...]], axis=1)
        cdims = (((1,), (1,)), ((), ()))
        gates = (lax.dot_general(xc, wih_ref[...], cdims,
                                 preferred_element_type=jnp.float32)
                 + bih_ref[...]
                 + lax.dot_general(h0_ref[...], whh_ref[...], cdims,
                                   preferred_element_type=jnp.float32)
                 + bhh_ref[...])
        H = LSTM_H
        gi = jax.nn.sigmoid(gates[:, 0:H])
        gf = jax.nn.sigmoid(gates[:, H:2 * H])
        gg = jnp.tanh(gates[:, 2 * H:3 * H])
        go = jax.nn.sigmoid(gates[:, 3 * H:4 * H])
        c1 = gf * c0_ref[...] + gi * gg
        h1 = go * jnp.tanh(c1)
        logits = lax.dot_general(h1, wfc_ref[...], cdims,
                                 preferred_element_type=jnp.float32) + bfc_ref[...]
        m = jnp.max(logits, axis=1, keepdims=True)
        lse = jnp.log(jnp.sum(jnp.exp(logits - m), axis=1, keepdims=True))
        xo_ref[...] = (logits - m - lse).reshape(1, 1, ACT_DIM)
        h1o_ref[...] = h1.reshape(1, 1, LSTM_H)
        c1o_ref[...] = c1.reshape(1, 1, LSTM_H)


_dense_c = pl.pallas_call(
    _dense_c_body,
    out_shape=[
        jax.ShapeDtypeStruct((1, 1, ACT_DIM), jnp.float32),
        jax.ShapeDtypeStruct((1, 1, LSTM_H), jnp.float32),
        jax.ShapeDtypeStruct((1, 1, LSTM_H), jnp.float32),
    ],
)


def kernel(x_graph, edge_index, edge_weight, x_state, h0, c0, W1, b1, W2, b2,
           W_ih, W_hh, b_ih, b_hh, W_fc, b_fc):
    npad = E_PAD - E
    em = 4 * edge_index - 9999 * (edge_index // NQ)
    pad_src = jnp.arange(npad, dtype=jnp.int32) % N
    pad_dst = jnp.arange(npad, dtype=jnp.int32) % (N_PAD - N) + N
    srcs = jnp.concatenate([em[0], pad_src]).reshape(NW, NCH, CHUNK)
    dsts = jnp.concatenate([em[1], pad_dst]).reshape(NW, NCH, CHUNK)
    ews = jnp.concatenate(
        [edge_weight, jnp.zeros((npad,), jnp.float32)]).reshape(NW, NCH, CHUNK)
    zeros1 = jnp.zeros((N_PAD,), jnp.float32)
    zeros2 = jnp.zeros((N_PAD, DH), jnp.float32)
    w2bd = jnp.kron(jnp.eye(4, dtype=jnp.float32), W2)

    degp = _deg_call(dsts, ews, zeros1)
    degp4 = degp.reshape(NC, N_PAD // 4, 4)
    h1 = _dense_a(x_graph, W1)
    accp1, norms = _msg1_call(h1.reshape(N, DH), srcs, dsts, ews, degp,
                              zeros2)
    h2 = _dense_b(accp1.reshape(NC, N_PAD // 4, 4 * DH), h1, degp4,
                  jnp.tile(b1.reshape(1, DH), (1, 4)), w2bd)
    accp2 = _msg2_call(h2.reshape(N, DH), srcs, dsts, norms, zeros2)
    xo, h1o, c1o = _dense_c(accp2.reshape(NC, N_PAD // 4, 4 * DH), h2, degp4,
                            jnp.tile(b2.reshape(1, DH), (1, 4)), x_state,
                            h0.reshape(1, LSTM_H), c0.reshape(1, LSTM_H),
                            W_ih, W_hh, b_ih.reshape(1, 4 * LSTM_H),
                            b_hh.reshape(1, 4 * LSTM_H), W_fc,
                            b_fc.reshape(1, ACT_DIM))
    return (xo, h1o, c1o)
